# Initial kernel scaffold; baseline (speedup 1.0000x reference)
import jax
import jax.numpy as jnp
from jax.experimental import pallas as pl


def kernel(M, N, K, emb_M, emb_N, emb_K, W1, b1, W2, b2):
    raise NotImplementedError("write your pallas kernel here")


# speedup vs baseline: 1.4425x; 1.4425x over previous
import functools

import jax
import jax.numpy as jnp
from jax import lax
from jax.experimental import pallas as pl
from jax.experimental.pallas import tpu as pltpu
from jax.experimental.pallas import tpu_sc as plsc

_B = 16384
_D = 16
_H = 128
_C = 387
_NC = 2
_NS = 16
_NW = _NC * _NS
_BPW = _B // _NW
_CH = 128
_NCHUNK = _BPW // _CH

_F32 = jnp.float32


def _sc_gather_body(m_idx, n_idx, k_idx, tbl_m, tbl_n, tbl_k,
                    out_m, out_n, out_k, idx_v, rows_v, sem):
    wid = lax.axis_index("s") * _NC + lax.axis_index("c")
    base = wid * _BPW
    for idx_hbm, tbl, out in ((m_idx, tbl_m, out_m),
                              (n_idx, tbl_n, out_n),
                              (k_idx, tbl_k, out_k)):
        pltpu.sync_copy(idx_hbm.at[pl.ds(base, _BPW)], idx_v)
        copies = []
        for j in range(_NCHUNK):
            copies.append(pltpu.async_copy(
                tbl.at[idx_v.at[pl.ds(j * _CH, _CH)]],
                rows_v.at[pl.ds(j * _CH, _CH)],
                sem))
        for cp in copies:
            cp.wait()
        pltpu.sync_copy(rows_v, out.at[pl.ds(base, _BPW)])


@functools.partial(jax.jit, static_argnums=())
def _sc_gather(m_i, n_i, k_i, tbl_m, tbl_n, tbl_k):
    f = pl.kernel(
        _sc_gather_body,
        out_type=(
            jax.ShapeDtypeStruct((_B, _D), _F32),
            jax.ShapeDtypeStruct((_B, _D), _F32),
            jax.ShapeDtypeStruct((_B, _D), _F32),
        ),
        mesh=plsc.VectorSubcoreMesh(core_axis_name="c", subcore_axis_name="s"),
        compiler_params=pltpu.CompilerParams(use_tc_tiling_on_sc=False),
        scratch_types=[
            pltpu.VMEM((_BPW,), jnp.int32),
            pltpu.VMEM((_BPW, _D), _F32),
            pltpu.SemaphoreType.DMA,
        ],
    )
    return f(m_i, n_i, k_i, tbl_m, tbl_n, tbl_k)


_BB = 2048


def _mlp_body(em_ref, en_ref, ek_ref, w1m_ref, w1n_ref, w1k_ref,
              b1_ref, w2_ref, b2_ref, out_ref):
    h = jnp.dot(em_ref[...], w1m_ref[...], preferred_element_type=_F32)
    h = h + jnp.dot(en_ref[...], w1n_ref[...], preferred_element_type=_F32)
    h = h + jnp.dot(ek_ref[...], w1k_ref[...], preferred_element_type=_F32)
    h = jnp.maximum(h + b1_ref[...], 0.0)
    logits = jnp.dot(h, w2_ref[...], preferred_element_type=_F32)
    logits = logits + b2_ref[...]
    m = jnp.max(logits, axis=1, keepdims=True)
    e = jnp.exp(logits - m)
    out_ref[...] = e / jnp.sum(e, axis=1, keepdims=True)


def _mlp(e_m, e_n, e_k, w1, b1, w2, b2):
    w1m, w1n, w1k = w1[0:_D], w1[_D:2 * _D], w1[2 * _D:3 * _D]
    b1r = b1.reshape(1, _H)
    b2r = b2.reshape(1, _C)
    grid = (_B // _BB,)
    e_spec = pl.BlockSpec((_BB, _D), lambda i: (i, 0))
    full = lambda shape: pl.BlockSpec(shape, lambda i: (0, 0))
    return pl.pallas_call(
        _mlp_body,
        grid=grid,
        in_specs=[
            e_spec, e_spec, e_spec,
            full((_D, _H)), full((_D, _H)), full((_D, _H)),
            full((1, _H)), full((_H, _C)), full((1, _C)),
        ],
        out_specs=pl.BlockSpec((_BB, _C), lambda i: (i, 0)),
        out_shape=jax.ShapeDtypeStruct((_B, _C), _F32),
    )(e_m, e_n, e_k, w1m, w1n, w1k, b1r, w2, b2r)


def kernel(M, N, K, emb_M, emb_N, emb_K, W1, b1, W2, b2):
    m_i = M.astype(jnp.int32)
    n_i = N.astype(jnp.int32)
    k_i = K.astype(jnp.int32)
    e_m, e_n, e_k = _sc_gather(m_i, n_i, k_i, emb_M, emb_N, emb_K)
    return _mlp(e_m, e_n, e_k, W1, b1, W2, b2)


# speedup vs baseline: 4.4658x; 3.0958x over previous
import functools

import jax
import jax.numpy as jnp
from jax import lax
from jax.experimental import pallas as pl
from jax.experimental.pallas import tpu as pltpu
from jax.experimental.pallas import tpu_sc as plsc

_B = 16384
_V = 100000
_D = 16
_H = 128
_C = 387
_NC = 2
_NS = 16
_CHUNK = 4096

_F32 = jnp.float32


def _serve_column(tbl, idx_hbm, col, row_off, base, n_rows,
                  col_v, idx_v, res_v, out):
    pltpu.sync_copy(tbl.at[col], col_v)
    for chunk in range(n_rows // _CHUNK):
        cbase = base + chunk * _CHUNK
        pltpu.sync_copy(idx_hbm.at[pl.ds(cbase, _CHUNK)], idx_v)

        @pl.loop(0, _CHUNK // 16, unroll=8)
        def _gather(i):
            ids = idx_v[pl.ds(i * 16, 16)]
            res_v[pl.ds(i * 16, 16)] = plsc.load_gather(col_v, [ids])

        pltpu.sync_copy(res_v, out.at[row_off + col, pl.ds(cbase, _CHUNK)])


def _sc_gather_body(m_idx, n_idx, k_idx, tbl_m, tbl_n, tbl_k, out,
                    col_v, idx_v, res_v):
    c = lax.axis_index("c")
    s = lax.axis_index("s")

    @pl.when(c == 0)
    def _():
        _serve_column(tbl_m, m_idx, s, 0, 0, _B, col_v, idx_v, res_v, out)

    @pl.when(c == 1)
    def _():
        _serve_column(tbl_n, n_idx, s, _D, 0, _B, col_v, idx_v, res_v, out)

    _serve_column(tbl_k, k_idx, s, 2 * _D, c * (_B // 2), _B // 2,
                  col_v, idx_v, res_v, out)


def _sc_gather(m_i, n_i, k_i, tbl_m_t, tbl_n_t, tbl_k_t):
    f = pl.kernel(
        _sc_gather_body,
        out_type=jax.ShapeDtypeStruct((3 * _D, _B), _F32),
        mesh=plsc.VectorSubcoreMesh(core_axis_name="c", subcore_axis_name="s"),
        compiler_params=pltpu.CompilerParams(needs_layout_passes=False),
        scratch_types=[
            pltpu.VMEM((_V,), _F32),
            pltpu.VMEM((_CHUNK,), jnp.int32),
            pltpu.VMEM((_CHUNK,), _F32),
        ],
    )
    return f(m_i, n_i, k_i, tbl_m_t, tbl_n_t, tbl_k_t)


_BB = 2048


def _mlp_body(cat_ref, w1t_ref, b1_ref, w2t_ref, b2_ref, out_ref):
    et = cat_ref[...]
    ht = jnp.dot(w1t_ref[...], et, preferred_element_type=_F32)
    ht = jnp.maximum(ht + b1_ref[...], 0.0)
    lt = jnp.dot(w2t_ref[...], ht, preferred_element_type=_F32)
    lt = lt + b2_ref[...]
    m = jnp.max(lt, axis=0, keepdims=True)
    e = jnp.exp(lt - m)
    out_ref[...] = e / jnp.sum(e, axis=0, keepdims=True)


def _mlp_t(cat_t, w1, b1, w2, b2):
    w1t = w1.T
    w2t = w2.T
    b1r = b1.reshape(_H, 1)
    b2r = b2.reshape(_C, 1)
    grid = (_B // _BB,)
    full = lambda shape: pl.BlockSpec(shape, lambda i: (0, 0))
    return pl.pallas_call(
        _mlp_body,
        grid=grid,
        in_specs=[
            pl.BlockSpec((3 * _D, _BB), lambda i: (0, i)),
            full((_H, 3 * _D)), full((_H, 1)),
            full((_C, _H)), full((_C, 1)),
        ],
        out_specs=pl.BlockSpec((_C, _BB), lambda i: (0, i)),
        out_shape=jax.ShapeDtypeStruct((_C, _B), _F32),
    )(cat_t, w1t, b1r, w2t, b2r)


def kernel(M, N, K, emb_M, emb_N, emb_K, W1, b1, W2, b2):
    m_i = M.astype(jnp.int32)
    n_i = N.astype(jnp.int32)
    k_i = K.astype(jnp.int32)
    cat_t = _sc_gather(m_i, n_i, k_i, emb_M.T, emb_N.T, emb_K.T)
    out_t = _mlp_t(cat_t, W1, b1, W2, b2)
    return out_t.T


# speedup vs baseline: 5.2739x; 1.1810x over previous
import functools

import jax
import jax.numpy as jnp
from jax import lax
from jax.experimental import pallas as pl
from jax.experimental.pallas import tpu as pltpu
from jax.experimental.pallas import tpu_sc as plsc

_B = 16384
_V = 100000
_D = 16
_H = 128
_C = 387
_NC = 2
_NS = 16
_CHUNK = 4096

_F32 = jnp.float32


def _serve_column(tbl, idx_hbm, col, row_off, base, n_rows,
                  col_v, idx_v, res_v, out):
    pltpu.sync_copy(tbl.at[col], col_v)
    for chunk in range(n_rows // _CHUNK):
        cbase = base + chunk * _CHUNK
        pltpu.sync_copy(idx_hbm.at[pl.ds(cbase, _CHUNK)], idx_v)

        @pl.loop(0, _CHUNK // 128)
        def _gather(i):
            base_w = i * 128
            ids = [idx_v[pl.ds(base_w + j * 16, 16)] for j in range(8)]
            vals = [plsc.load_gather(col_v, [v]) for v in ids]
            for j, v in enumerate(vals):
                res_v[pl.ds(base_w + j * 16, 16)] = v

        pltpu.sync_copy(res_v, out.at[row_off + col, pl.ds(cbase, _CHUNK)])


def _sc_gather_body(m_idx, n_idx, k_idx, tbl_m, tbl_n, tbl_k, out,
                    col_v, idx_v, res_v):
    c = lax.axis_index("c")
    s = lax.axis_index("s")

    @pl.when(c == 0)
    def _():
        _serve_column(tbl_m, m_idx, s, 0, 0, _B, col_v, idx_v, res_v, out)

    @pl.when(c == 1)
    def _():
        _serve_column(tbl_n, n_idx, s, _D, 0, _B, col_v, idx_v, res_v, out)

    _serve_column(tbl_k, k_idx, s, 2 * _D, c * (_B // 2), _B // 2,
                  col_v, idx_v, res_v, out)


def _sc_gather(m_i, n_i, k_i, tbl_m_t, tbl_n_t, tbl_k_t):
    f = pl.kernel(
        _sc_gather_body,
        out_type=jax.ShapeDtypeStruct((3 * _D, _B), _F32),
        mesh=plsc.VectorSubcoreMesh(core_axis_name="c", subcore_axis_name="s"),
        compiler_params=pltpu.CompilerParams(needs_layout_passes=False),
        scratch_types=[
            pltpu.VMEM((_V,), _F32),
            pltpu.VMEM((_CHUNK,), jnp.int32),
            pltpu.VMEM((_CHUNK,), _F32),
        ],
    )
    return f(m_i, n_i, k_i, tbl_m_t, tbl_n_t, tbl_k_t)


_BB = 2048


def _mlp_body(cat_ref, w1t_ref, b1_ref, w2t_ref, b2_ref, out_ref):
    et = cat_ref[...]
    ht = jnp.dot(w1t_ref[...], et, preferred_element_type=_F32)
    ht = jnp.maximum(ht + b1_ref[...], 0.0)
    lt = jnp.dot(w2t_ref[...], ht, preferred_element_type=_F32)
    lt = lt + b2_ref[...]
    m = jnp.max(lt, axis=0, keepdims=True)
    e = jnp.exp(lt - m)
    out_ref[...] = e / jnp.sum(e, axis=0, keepdims=True)


def _mlp_t(cat_t, w1, b1, w2, b2):
    w1t = w1.T
    w2t = w2.T
    b1r = b1.reshape(_H, 1)
    b2r = b2.reshape(_C, 1)
    grid = (_B // _BB,)
    full = lambda shape: pl.BlockSpec(shape, lambda i: (0, 0))
    return pl.pallas_call(
        _mlp_body,
        grid=grid,
        in_specs=[
            pl.BlockSpec((3 * _D, _BB), lambda i: (0, i)),
            full((_H, 3 * _D)), full((_H, 1)),
            full((_C, _H)), full((_C, 1)),
        ],
        out_specs=pl.BlockSpec((_C, _BB), lambda i: (0, i)),
        out_shape=jax.ShapeDtypeStruct((_C, _B), _F32),
    )(cat_t, w1t, b1r, w2t, b2r)


def kernel(M, N, K, emb_M, emb_N, emb_K, W1, b1, W2, b2):
    m_i = M.astype(jnp.int32)
    n_i = N.astype(jnp.int32)
    k_i = K.astype(jnp.int32)
    cat_t = _sc_gather(m_i, n_i, k_i, emb_M.T, emb_N.T, emb_K.T)
    out_t = _mlp_t(cat_t, W1, b1, W2, b2)
    return out_t.T


# speedup vs baseline: 5.7330x; 1.0871x over previous
import functools

import jax
import jax.numpy as jnp
from jax import lax
from jax.experimental import pallas as pl
from jax.experimental.pallas import tpu as pltpu
from jax.experimental.pallas import tpu_sc as plsc

_B = 16384
_V = 100000
_D = 16
_H = 128
_C = 387
_NC = 2
_NS = 16
_CHUNK = 4096

_F32 = jnp.float32


def _serve_column(tbl, idx_hbm, col, row_off, base, n_rows,
                  col_v, idx_v, res_v, sem_col, sem_idx, sem_out, out):
    cp_idx = pltpu.async_copy(idx_hbm.at[pl.ds(base, n_rows)],
                              idx_v.at[pl.ds(0, n_rows)], sem_idx)
    cp_col = pltpu.async_copy(tbl.at[col], col_v, sem_col)
    cp_idx.wait()
    cp_col.wait()
    out_cps = []
    for chunk in range(n_rows // _CHUNK):
        half = (chunk % 2) * _CHUNK
        if chunk >= 2:
            out_cps[chunk - 2].wait()

        @pl.loop(0, _CHUNK // 128)
        def _gather(i):
            base_w = chunk * _CHUNK + i * 128
            res_w = half + i * 128
            ids = [idx_v[pl.ds(base_w + j * 16, 16)] for j in range(8)]
            vals = [plsc.load_gather(col_v, [v]) for v in ids]
            for j, v in enumerate(vals):
                res_v[pl.ds(res_w + j * 16, 16)] = v

        out_cps.append(pltpu.async_copy(
            res_v.at[pl.ds(half, _CHUNK)],
            out.at[row_off + col, pl.ds(base + chunk * _CHUNK, _CHUNK)],
            sem_out))
    for cp in out_cps[-2:]:
        cp.wait()


def _sc_gather_body(m_idx, n_idx, k_idx, tbl_m, tbl_n, tbl_k, out,
                    col_v, idx_v, res_v, sem_col, sem_idx, sem_out):
    c = lax.axis_index("c")
    s = lax.axis_index("s")
    sems = (sem_col, sem_idx, sem_out)

    @pl.when(c == 0)
    def _():
        _serve_column(tbl_m, m_idx, s, 0, 0, _B, col_v, idx_v, res_v,
                      *sems, out)

    @pl.when(c == 1)
    def _():
        _serve_column(tbl_n, n_idx, s, _D, 0, _B, col_v, idx_v, res_v,
                      *sems, out)

    _serve_column(tbl_k, k_idx, s, 2 * _D, c * (_B // 2), _B // 2,
                  col_v, idx_v, res_v, *sems, out)


def _sc_gather(m_i, n_i, k_i, tbl_m_t, tbl_n_t, tbl_k_t):
    f = pl.kernel(
        _sc_gather_body,
        out_type=jax.ShapeDtypeStruct((3 * _D, _B), _F32),
        mesh=plsc.VectorSubcoreMesh(core_axis_name="c", subcore_axis_name="s"),
        compiler_params=pltpu.CompilerParams(needs_layout_passes=False),
        scratch_types=[
            pltpu.VMEM((_V,), _F32),
            pltpu.VMEM((_B,), jnp.int32),
            pltpu.VMEM((2 * _CHUNK,), _F32),
            pltpu.SemaphoreType.DMA,
            pltpu.SemaphoreType.DMA,
            pltpu.SemaphoreType.DMA,
        ],
    )
    return f(m_i, n_i, k_i, tbl_m_t, tbl_n_t, tbl_k_t)


_BB = 2048


def _mlp_body(cat_ref, w1t_ref, b1_ref, w2t_ref, b2_ref, out_ref):
    et = cat_ref[...]
    ht = jnp.dot(w1t_ref[...], et, preferred_element_type=_F32)
    ht = jnp.maximum(ht + b1_ref[...], 0.0)
    lt = jnp.dot(w2t_ref[...], ht, preferred_element_type=_F32)
    lt = lt + b2_ref[...]
    m = jnp.max(lt, axis=0, keepdims=True)
    e = jnp.exp(lt - m)
    out_ref[...] = e / jnp.sum(e, axis=0, keepdims=True)


def _mlp_t(cat_t, w1, b1, w2, b2):
    w1t = w1.T
    w2t = w2.T
    b1r = b1.reshape(_H, 1)
    b2r = b2.reshape(_C, 1)
    grid = (_B // _BB,)
    full = lambda shape: pl.BlockSpec(shape, lambda i: (0, 0))
    return pl.pallas_call(
        _mlp_body,
        grid=grid,
        in_specs=[
            pl.BlockSpec((3 * _D, _BB), lambda i: (0, i)),
            full((_H, 3 * _D)), full((_H, 1)),
            full((_C, _H)), full((_C, 1)),
        ],
        out_specs=pl.BlockSpec((_C, _BB), lambda i: (0, i)),
        out_shape=jax.ShapeDtypeStruct((_C, _B), _F32),
    )(cat_t, w1t, b1r, w2t, b2r)


def kernel(M, N, K, emb_M, emb_N, emb_K, W1, b1, W2, b2):
    m_i = M.astype(jnp.int32)
    n_i = N.astype(jnp.int32)
    k_i = K.astype(jnp.int32)
    cat_t = _sc_gather(m_i, n_i, k_i, emb_M.T, emb_N.T, emb_K.T)
    out_t = _mlp_t(cat_t, W1, b1, W2, b2)
    return out_t.T
